# baseline (speedup 1.0000x reference)
import jax
import jax.numpy as jnp
from jax import lax
from jax.experimental import pallas as pl
from jax.experimental.pallas import tpu as pltpu
from jax.experimental.pallas import tpu_sc as plsc

D_MODEL = 64
LANES = 16
NUM_CORES = 2
NUM_SUBCORES = 16
NUM_WORKERS = NUM_CORES * NUM_SUBCORES
SCALE = 1.0 / 8.0

VOCAB = 1000000
SB_COLS = 256
N_SB = VOCAB // SB_COLS
SB_PER_W = (N_SB + NUM_WORKERS - 1) // NUM_WORKERS
TAIL_COLS = VOCAB - N_SB * SB_COLS
SKEW_A = SB_COLS + 1
SKEW_B = 129
N_BUF_B = 4
PRE_B = 3


def _iota16():
    return lax.iota(jnp.int32, LANES)


def _pack_body(tt_hbm, tp_hbm, abuf, pbuf, cbuf, isems, osems):
    wid = lax.axis_index("s") * NUM_CORES + lax.axis_index("c")

    def fire_in(sb, p):
        pltpu.async_copy(
            tt_hbm.at[:, pl.ds(sb * SB_COLS, SB_COLS)], abuf.at[p], isems[p]
        )

    def transpose_block(src2d, dst2d, ncols):
        for cb in range(ncols // LANES):
            c0 = cb * LANES
            cvec = c0 + _iota16()
            crow = lax.shift_right_logical(cvec, 1)
            codd = lax.shift_left(lax.bitwise_and(cvec, 1), 6)

            @plsc.parallel_loop(0, LANES, unroll=4)
            def diag(j, _crow=crow, _codd=codd, _cvec=cvec):
                dj = lax.bitwise_and(_iota16() + j, LANES - 1)
                for f0 in range(0, D_MODEL, LANES):
                    v = plsc.load_gather(src2d, [f0 + dj, _cvec])
                    plsc.store_scatter(dst2d, [_crow, _codd + f0 + dj], v * SCALE)

    sb0 = wid

    @pl.when(sb0 < N_SB)
    def _():
        fire_in(sb0, 0)

    def step(i2, carry):
        for p in range(2):
            i = i2 * 2 + p
            sb = wid + i * NUM_WORKERS

            @pl.when(sb < N_SB)
            def _():
                @pl.when(sb + NUM_WORKERS < N_SB)
                def _():
                    fire_in(sb + NUM_WORKERS, 1 - p)

                pltpu.make_async_copy(
                    tt_hbm.at[:, pl.ds(0, SB_COLS)], abuf.at[p], isems[p]
                ).wait()

                @pl.when(i >= 2)
                def _():
                    pltpu.make_async_copy(
                        pbuf.at[p], tp_hbm.at[pl.ds(0, SB_COLS // 2)], osems[p]
                    ).wait()

                transpose_block(abuf.at[p], pbuf.at[p], SB_COLS)
                pltpu.async_copy(
                    pbuf.at[p],
                    tp_hbm.at[pl.ds(sb * (SB_COLS // 2), SB_COLS // 2)],
                    osems[p],
                )

        return carry

    lax.fori_loop(0, (SB_PER_W + 1) // 2, step, 0)

    n_mine = (N_SB - wid + NUM_WORKERS - 1) // NUM_WORKERS

    for p in range(2):
        @pl.when(n_mine >= p + 1)
        def _():
            pltpu.make_async_copy(
                pbuf.at[p], tp_hbm.at[pl.ds(0, SB_COLS // 2)], osems[p]
            ).wait()

    @pl.when(wid == 0)
    def _():
        pltpu.sync_copy(tt_hbm.at[:, pl.ds(N_SB * SB_COLS, TAIL_COLS)], cbuf)

        transpose_block(cbuf, pbuf.at[0], TAIL_COLS)

        pltpu.sync_copy(
            pbuf.at[0, pl.ds(0, TAIL_COLS // 2)],
            tp_hbm.at[pl.ds(N_SB * (SB_COLS // 2), TAIL_COLS // 2)],
        )


def _gather_body(xt_hbm, tp_hbm, out_hbm, idx_all, jbuf, gbuf, tbuf, sbuf, gsems, osems):
    wid = lax.axis_index("s") * NUM_CORES + lax.axis_index("c")
    n_units = xt_hbm.shape[0]
    pltpu.sync_copy(xt_hbm.at[:, pl.ds(wid * 128, 128)], idx_all)

    def fire(u, p):
        for g in range(128 // LANES):
            iv = idx_all[u, pl.ds(g * LANES, LANES)]
            jbuf[p, pl.ds(g * LANES, LANES)] = lax.shift_right_logical(iv, 1)
        pltpu.async_copy(tp_hbm.at[jbuf.at[p]], gbuf.at[p], gsems[p])

    for u0 in range(PRE_B):
        fire(u0, u0)

    def step(i, carry):
        for k in range(N_BUF_B):
            u = i * N_BUF_B + k
            p = k
            t = k % 2
            pltpu.make_async_copy(
                tp_hbm.at[pl.ds(0, 128)], gbuf.at[p], gsems[p]
            ).wait()

            @pl.when(u + PRE_B < n_units)
            def _():
                fire(u + PRE_B, (k + PRE_B) % N_BUF_B)

            @pl.when(u >= 2)
            def _():
                pltpu.make_async_copy(
                    tbuf.at[t], out_hbm.at[0, :, pl.ds(0, 128)], osems[t]
                ).wait()

            for g in range(128 // LANES):
                iv = idx_all[u, pl.ds(g * LANES, LANES)]
                h64 = lax.shift_left(lax.bitwise_and(iv, 1), 6)
                rows = _iota16() + g * LANES

                @plsc.parallel_loop(0, LANES, unroll=4)
                def diag(j, _h64=h64, _rows=rows, _g=g, _t=t, _p=p):
                    dj = lax.bitwise_and(_iota16() + j, LANES - 1)
                    for f0 in range(0, D_MODEL, LANES):
                        v = plsc.load_gather(
                            gbuf.at[_p], [_rows, _h64 + f0 + dj]
                        )
                        plsc.store_scatter(
                            tbuf.at[_t], [f0 + dj, _rows], v
                        )

            pltpu.async_copy(
                tbuf.at[t], out_hbm.at[u, :, pl.ds(wid * 128, 128)], osems[t]
            )
        return carry

    lax.fori_loop(0, n_units // N_BUF_B, step, 0)

    for t in range(2):
        pltpu.make_async_copy(
            tbuf.at[t], out_hbm.at[0, :, pl.ds(0, 128)], osems[t]
        ).wait()


@jax.jit
def kernel(x, table):
    b, s = x.shape
    tt = table.T
    xt = x.T

    pack = pl.kernel(
        _pack_body,
        out_type=jax.ShapeDtypeStruct((VOCAB // 2, 2 * D_MODEL), jnp.float32),
        mesh=plsc.VectorSubcoreMesh(core_axis_name="c", subcore_axis_name="s"),
        compiler_params=pltpu.CompilerParams(needs_layout_passes=False),
        scratch_types=[
            pltpu.VMEM((2, D_MODEL, SB_COLS), jnp.float32),
            pltpu.VMEM((2, SB_COLS // 2, 2 * D_MODEL), jnp.float32),
            pltpu.VMEM((D_MODEL, TAIL_COLS), jnp.float32),
            [pltpu.SemaphoreType.DMA] * 2,
            [pltpu.SemaphoreType.DMA] * 2,
        ],
    )
    tp = pack(tt)

    gather = pl.kernel(
        _gather_body,
        out_type=jax.ShapeDtypeStruct((s, D_MODEL, b), jnp.float32),
        mesh=plsc.VectorSubcoreMesh(core_axis_name="c", subcore_axis_name="s"),
        compiler_params=pltpu.CompilerParams(needs_layout_passes=False),
        scratch_types=[
            pltpu.VMEM((s, 128), jnp.int32),
            pltpu.VMEM((N_BUF_B, 128), jnp.int32),
            pltpu.VMEM((N_BUF_B, 128, 2 * D_MODEL), jnp.float32),
            pltpu.VMEM((2, D_MODEL, 128), jnp.float32),
            pltpu.VMEM((128 * SKEW_B,), jnp.float32),
            [pltpu.SemaphoreType.DMA] * N_BUF_B,
            [pltpu.SemaphoreType.DMA] * 2,
        ],
    )
    o = gather(xt, tp)
    return jnp.transpose(o, (2, 0, 1))


# speedup vs baseline: 1.4287x; 1.4287x over previous
import jax
import jax.numpy as jnp
from jax import lax
from jax.experimental import pallas as pl
from jax.experimental.pallas import tpu as pltpu
from jax.experimental.pallas import tpu_sc as plsc

D_MODEL = 64
LANES = 16
NUM_CORES = 2
NUM_SUBCORES = 16
NUM_WORKERS = NUM_CORES * NUM_SUBCORES
SCALE = 1.0 / 8.0

VOCAB = 1000000
SB_COLS = 256
N_SB = VOCAB // SB_COLS
SB_PER_W = (N_SB + NUM_WORKERS - 1) // NUM_WORKERS
TAIL_COLS = VOCAB - N_SB * SB_COLS
SKEW_A = SB_COLS + 1
N_BUF_B = 4
PRE_B = 3


def _iota16():
    return lax.iota(jnp.int32, LANES)


def _pack_body(tt_hbm, tp_hbm, abuf, pbuf, cbuf, sbuf, isems, osems):
    wid = lax.axis_index("s") * NUM_CORES + lax.axis_index("c")

    def fire_in(sb, p):
        pltpu.async_copy(
            tt_hbm.at[:, pl.ds(sb * SB_COLS, SB_COLS)], abuf.at[p], isems[p]
        )

    def transpose_block(p, ncols):
        @plsc.parallel_loop(0, D_MODEL, unroll=8)
        def stage(f):
            for cg in range(ncols // LANES):
                sbuf[pl.ds(f * SKEW_A + cg * LANES, LANES)] = abuf[
                    p, f, pl.ds(cg * LANES, LANES)
                ]

        bases = [(_iota16() + f0) * SKEW_A for f0 in range(0, D_MODEL, LANES)]

        @plsc.parallel_loop(0, ncols // 2, unroll=4)
        def row(jj):
            for h in range(2):
                c = 2 * jj + h
                for q in range(D_MODEL // LANES):
                    v = plsc.load_gather(sbuf, [bases[q] + c])
                    pbuf[p, jj, pl.ds(h * D_MODEL + q * LANES, LANES)] = v * SCALE

    sb0 = wid

    @pl.when(sb0 < N_SB)
    def _():
        fire_in(sb0, 0)

    def step(i2, carry):
        for p in range(2):
            i = i2 * 2 + p
            sb = wid + i * NUM_WORKERS

            @pl.when(sb < N_SB)
            def _():
                @pl.when(sb + NUM_WORKERS < N_SB)
                def _():
                    fire_in(sb + NUM_WORKERS, 1 - p)

                pltpu.make_async_copy(
                    tt_hbm.at[:, pl.ds(0, SB_COLS)], abuf.at[p], isems[p]
                ).wait()

                @pl.when(i >= 2)
                def _():
                    pltpu.make_async_copy(
                        pbuf.at[p], tp_hbm.at[pl.ds(0, SB_COLS // 2)], osems[p]
                    ).wait()

                transpose_block(p, SB_COLS)
                pltpu.async_copy(
                    pbuf.at[p],
                    tp_hbm.at[pl.ds(sb * (SB_COLS // 2), SB_COLS // 2)],
                    osems[p],
                )

        return carry

    lax.fori_loop(0, (SB_PER_W + 1) // 2, step, 0)

    n_mine = (N_SB - wid + NUM_WORKERS - 1) // NUM_WORKERS

    for p in range(2):
        @pl.when(n_mine >= p + 1)
        def _():
            pltpu.make_async_copy(
                pbuf.at[p], tp_hbm.at[pl.ds(0, SB_COLS // 2)], osems[p]
            ).wait()

    @pl.when(wid == 0)
    def _():
        pltpu.sync_copy(tt_hbm.at[:, pl.ds(N_SB * SB_COLS, TAIL_COLS)], cbuf)

        @plsc.parallel_loop(0, D_MODEL, unroll=8)
        def stage(f):
            for cg in range(TAIL_COLS // LANES):
                sbuf[pl.ds(f * SKEW_A + cg * LANES, LANES)] = cbuf[
                    f, pl.ds(cg * LANES, LANES)
                ]

        bases = [(_iota16() + f0) * SKEW_A for f0 in range(0, D_MODEL, LANES)]

        @plsc.parallel_loop(0, TAIL_COLS // 2, unroll=4)
        def row(jj):
            for h in range(2):
                cc = 2 * jj + h
                for q in range(D_MODEL // LANES):
                    v = plsc.load_gather(sbuf, [bases[q] + cc])
                    pbuf[0, jj, pl.ds(h * D_MODEL + q * LANES, LANES)] = v * SCALE
        pltpu.sync_copy(
            pbuf.at[0, pl.ds(0, TAIL_COLS // 2)],
            tp_hbm.at[pl.ds(N_SB * (SB_COLS // 2), TAIL_COLS // 2)],
        )


def _gather_body(xt_hbm, tp_hbm, out_hbm, idx_all, jbuf, gbuf, tbuf, gsems, osems):
    wid = lax.axis_index("s") * NUM_CORES + lax.axis_index("c")
    n_units = xt_hbm.shape[0]
    pltpu.sync_copy(xt_hbm.at[:, pl.ds(wid * 128, 128)], idx_all)

    def fire(u, p):
        for g in range(128 // LANES):
            iv = idx_all[u, pl.ds(g * LANES, LANES)]
            jbuf[p, pl.ds(g * LANES, LANES)] = lax.shift_right_logical(iv, 1)
        pltpu.async_copy(tp_hbm.at[jbuf.at[p]], gbuf.at[p], gsems[p])

    for u0 in range(PRE_B):
        fire(u0, u0)

    def step(i, carry):
        for k in range(N_BUF_B):
            u = i * N_BUF_B + k
            p = k
            t = k % 2
            pltpu.make_async_copy(
                tp_hbm.at[pl.ds(0, 128)], gbuf.at[p], gsems[p]
            ).wait()

            @pl.when(u + PRE_B < n_units)
            def _():
                fire(u + PRE_B, (k + PRE_B) % N_BUF_B)

            @pl.when(u >= 2)
            def _():
                pltpu.make_async_copy(
                    tbuf.at[t], out_hbm.at[0, :, pl.ds(0, 128)], osems[t]
                ).wait()

            for g in range(128 // LANES):
                iv = idx_all[u, pl.ds(g * LANES, LANES)]
                h64 = lax.shift_left(lax.bitwise_and(iv, 1), 6)
                rows = _iota16() + g * LANES

                @plsc.parallel_loop(0, LANES, unroll=4)
                def diag(j, _h64=h64, _rows=rows, _g=g, _t=t, _p=p):
                    dj = lax.bitwise_and(_iota16() + j, LANES - 1)
                    for f0 in range(0, D_MODEL, LANES):
                        v = plsc.load_gather(
                            gbuf.at[_p], [_rows, _h64 + f0 + dj]
                        )
                        plsc.store_scatter(
                            tbuf.at[_t], [f0 + dj, _rows], v
                        )

            pltpu.async_copy(
                tbuf.at[t], out_hbm.at[u, :, pl.ds(wid * 128, 128)], osems[t]
            )
        return carry

    lax.fori_loop(0, n_units // N_BUF_B, step, 0)

    for t in range(2):
        pltpu.make_async_copy(
            tbuf.at[t], out_hbm.at[0, :, pl.ds(0, 128)], osems[t]
        ).wait()


@jax.jit
def kernel(x, table):
    b, s = x.shape
    tt = table.T
    xt = x.T

    pack = pl.kernel(
        _pack_body,
        out_type=jax.ShapeDtypeStruct((VOCAB // 2, 2 * D_MODEL), jnp.float32),
        mesh=plsc.VectorSubcoreMesh(core_axis_name="c", subcore_axis_name="s"),
        compiler_params=pltpu.CompilerParams(needs_layout_passes=False),
        scratch_types=[
            pltpu.VMEM((2, D_MODEL, SB_COLS), jnp.float32),
            pltpu.VMEM((2, SB_COLS // 2, 2 * D_MODEL), jnp.float32),
            pltpu.VMEM((D_MODEL, TAIL_COLS), jnp.float32),
            pltpu.VMEM((D_MODEL * SKEW_A,), jnp.float32),
            [pltpu.SemaphoreType.DMA] * 2,
            [pltpu.SemaphoreType.DMA] * 2,
        ],
    )
    tp = pack(tt)

    gather = pl.kernel(
        _gather_body,
        out_type=jax.ShapeDtypeStruct((s, D_MODEL, b), jnp.float32),
        mesh=plsc.VectorSubcoreMesh(core_axis_name="c", subcore_axis_name="s"),
        compiler_params=pltpu.CompilerParams(needs_layout_passes=False),
        scratch_types=[
            pltpu.VMEM((s, 128), jnp.int32),
            pltpu.VMEM((N_BUF_B, 128), jnp.int32),
            pltpu.VMEM((N_BUF_B, 128, 2 * D_MODEL), jnp.float32),
            pltpu.VMEM((2, D_MODEL, 128), jnp.float32),
            [pltpu.SemaphoreType.DMA] * N_BUF_B,
            [pltpu.SemaphoreType.DMA] * 2,
        ],
    )
    o = gather(xt, tp)
    return jnp.transpose(o, (2, 0, 1))
